# baseline (speedup 1.0000x reference)
import functools

import jax
import jax.numpy as jnp
from jax import lax
from jax.experimental import pallas as pl
from jax.experimental.pallas import tpu as pltpu
from jax.experimental.pallas import tpu_sc as plsc

B = 4
T = 2048
D = 64
N = B * T
NC = 2
NS = 16
NW = NC * NS
PER_W = N // NW
LANES = 16
CPW = PER_W // LANES
UTOK = 64
NU = PER_W // UTOK
VPR = D // LANES

VOCAB = 1_000_000


def _emb_body(idx_hbm, par_hbm, table_hbm, pos_hbm, out_hbm,
              idx_v, row_v, parf_v, gath_v, pos_v, blk_v,
              sem0, sem1, psem):
    wid = lax.axis_index("s") * NC + lax.axis_index("c")
    base = wid * PER_W
    tbase = lax.rem(base, T)

    pltpu.sync_copy(idx_hbm.at[wid], idx_v)
    pltpu.sync_copy(par_hbm.at[wid], parf_v)

    pos_cp = pltpu.async_copy(pos_hbm.at[pl.ds(tbase, PER_W)], pos_v, psem)

    def build(c, carry):
        sl = pl.ds(c * LANES, LANES)
        row_v[0, sl] = idx_v[0, sl] >> 1
        return carry

    lax.fori_loop(0, CPW, build, 0)

    sems = (sem0, sem1)
    cps = {}

    def fire(u, slot):
        cps[u] = pltpu.async_copy(
            table_hbm.at[row_v.at[0, pl.ds(u * UTOK, UTOK)]],
            blk_v.at[slot],
            sems[slot],
        )

    fire(0, 0)
    fire(1, 1)
    pos_cp.wait()

    for u in range(NU):
        slot = u & 1
        cps[u].wait()

        def select(j, carry):
            jj = u * UTOK + j
            parf = parf_v[jj, pl.ds(0, LANES)]
            for c in range(VPR):
                csl = pl.ds(c * LANES, LANES)
                lo = blk_v[slot, j, csl]
                hi = blk_v[slot, j, pl.ds(D + c * LANES, LANES)]
                gath_v[jj, csl] = lo + (hi - lo) * parf + pos_v[jj, csl]
            return carry

        lax.fori_loop(0, UTOK, select, 0)
        if u + 2 < NU:
            fire(u + 2, slot)

    pltpu.sync_copy(gath_v, out_hbm.at[pl.ds(base, PER_W), :])


@functools.cache
def _emb_kernel():
    mesh = plsc.VectorSubcoreMesh(core_axis_name="c", subcore_axis_name="s")
    return pl.kernel(
        _emb_body,
        mesh=mesh,
        out_type=jax.ShapeDtypeStruct((N, D), jnp.float32),
        scratch_types=[
            pltpu.VMEM((1, PER_W), jnp.int32),
            pltpu.VMEM((1, PER_W), jnp.int32),
            pltpu.VMEM((PER_W, LANES), jnp.float32),
            pltpu.VMEM((PER_W, D), jnp.float32),
            pltpu.VMEM((PER_W, D), jnp.float32),
            pltpu.VMEM((2, UTOK, 2 * D), jnp.float32),
            pltpu.SemaphoreType.DMA,
            pltpu.SemaphoreType.DMA,
            pltpu.SemaphoreType.DMA,
        ],
    )


def kernel(x, embedding_table, pos_table):
    xi = x.astype(jnp.int32)
    idx = xi.reshape(NW, 1, PER_W)
    par = jnp.broadcast_to(
        (xi & 1).astype(jnp.float32).reshape(NW, PER_W, 1), (NW, PER_W, LANES)
    )
    pairs = embedding_table.reshape(VOCAB // 2, 2 * D)
    out = _emb_kernel()(idx, par, pairs, pos_table)
    return out.reshape(B, T, D)
